# Initial kernel scaffold; baseline (speedup 1.0000x reference)
import jax
import jax.numpy as jnp
from jax.experimental import pallas as pl


def kernel(input_ids, table, W, b):
    raise NotImplementedError("write your pallas kernel here")


# speedup vs baseline: 1.3782x; 1.3782x over previous
import functools

import jax
import jax.numpy as jnp
from jax import lax
from jax.experimental import pallas as pl
from jax.experimental.pallas import tpu as pltpu
from jax.experimental.pallas import tpu_sc as plsc

NC = 2
NS = 16
NW = NC * NS


def _pool_body(seq, hidden, bpw, ids_hbm, table_hbm, x_hbm,
               idx_v, rows0, rows1, xbuf, sem0, sem1):
    half = seq // 2
    wid = lax.axis_index("s") * NC + lax.axis_index("c")
    base = wid * bpw
    pltpu.sync_copy(ids_hbm.at[pl.ds(2 * base, 2 * bpw)], idx_v)

    bufs = (rows0, rows1)
    sems = (sem0, sem1)

    def start_gather(b):
        buf = bufs[b % 2]
        sem = sems[b % 2]
        c0 = pltpu.async_copy(table_hbm.at[idx_v.at[2 * b]],
                              buf.at[pl.ds(0, half)], sem)
        c1 = pltpu.async_copy(table_hbm.at[idx_v.at[2 * b + 1]],
                              buf.at[pl.ds(half, half)], sem)
        return (c0, c1)

    pending = [None, None]
    pending[0] = start_gather(0)
    scale = jnp.float32(1.0 / seq)
    for b in range(bpw):
        if b + 1 < bpw:
            pending[(b + 1) % 2] = start_gather(b + 1)
        for c in pending[b % 2]:
            c.wait()
        rows = bufs[b % 2]

        def body(s, accs):
            a0, a1, a2, a3 = accs
            a0 = a0 + rows[s, pl.ds(0, 16)]
            a1 = a1 + rows[s, pl.ds(16, 16)]
            a2 = a2 + rows[s, pl.ds(32, 16)]
            a3 = a3 + rows[s, pl.ds(48, 16)]
            return (a0, a1, a2, a3)

        z = jnp.zeros((16,), jnp.float32)
        a0, a1, a2, a3 = lax.fori_loop(0, seq, body, (z, z, z, z))
        xbuf[b, pl.ds(0, 16)] = a0 * scale
        xbuf[b, pl.ds(16, 16)] = a1 * scale
        xbuf[b, pl.ds(32, 16)] = a2 * scale
        xbuf[b, pl.ds(48, 16)] = a3 * scale
    pltpu.sync_copy(xbuf, x_hbm.at[pl.ds(base, bpw)])


def _pool_sc(input_ids, table):
    batch, seq = input_ids.shape
    vocab, hidden = table.shape
    bpw = batch // NW
    ids2 = input_ids.reshape(2 * batch, seq // 2)
    mesh = plsc.VectorSubcoreMesh(core_axis_name="c", subcore_axis_name="s")
    fn = pl.kernel(
        functools.partial(_pool_body, seq, hidden, bpw),
        out_type=jax.ShapeDtypeStruct((batch, hidden), jnp.float32),
        mesh=mesh,
        scratch_types=[
            pltpu.VMEM((2 * bpw, seq // 2), jnp.int32),
            pltpu.VMEM((seq, hidden), jnp.float32),
            pltpu.VMEM((seq, hidden), jnp.float32),
            pltpu.VMEM((bpw, hidden), jnp.float32),
            pltpu.SemaphoreType.DMA,
            pltpu.SemaphoreType.DMA,
        ],
        compiler_params=pltpu.CompilerParams(use_tc_tiling_on_sc=False),
    )
    return fn(ids2, table)


def _mm_body(x_ref, w_ref, b_ref, out_ref):
    out_ref[...] = lax.dot_general(
        x_ref[...], w_ref[...],
        dimension_numbers=(((1,), (1,)), ((), ())),
        preferred_element_type=jnp.float32,
    ) + b_ref[...]


def _linear_tc(x, W, b, vb=1024):
    batch, hidden = x.shape
    vocab = W.shape[0]
    grid = (pl.cdiv(vocab, vb),)
    return pl.pallas_call(
        _mm_body,
        grid=grid,
        in_specs=[
            pl.BlockSpec((batch, hidden), lambda j: (0, 0)),
            pl.BlockSpec((vb, hidden), lambda j: (j, 0)),
            pl.BlockSpec((1, vb), lambda j: (0, j)),
        ],
        out_specs=pl.BlockSpec((batch, vb), lambda j: (0, j)),
        out_shape=jax.ShapeDtypeStruct((batch, vocab), jnp.float32),
        compiler_params=pltpu.CompilerParams(
            dimension_semantics=("parallel",)),
    )(x, W, b.reshape(1, vocab))


def kernel(input_ids, table, W, b):
    x = _pool_sc(input_ids, table)
    logits = _linear_tc(x, W, b)
    return (logits, x)


# speedup vs baseline: 1.4326x; 1.0395x over previous
import functools

import jax
import jax.numpy as jnp
from jax import lax
from jax.experimental import pallas as pl
from jax.experimental.pallas import tpu as pltpu
from jax.experimental.pallas import tpu_sc as plsc

NC = 2
NS = 16
NW = NC * NS


def _pool_body(seq, hidden, bpw, ids_hbm, table_hbm, x_hbm,
               idx_v, rows0, rows1, xbuf, sem0, sem1):
    half = seq // 2
    wid = lax.axis_index("s") * NC + lax.axis_index("c")
    base = wid * bpw
    pltpu.sync_copy(ids_hbm.at[pl.ds(base, bpw)], idx_v)

    bufs = (rows0, rows1)
    sems = (sem0, sem1)

    def start_gather(b):
        buf = bufs[b % 2]
        sem = sems[b % 2]
        w0 = min(128, seq)
        c0 = pltpu.async_copy(table_hbm.at[idx_v.at[b, pl.ds(0, w0)]],
                              buf.at[pl.ds(0, w0)], sem)
        c1 = pltpu.async_copy(table_hbm.at[idx_v.at[b, pl.ds(w0, seq - w0)]],
                              buf.at[pl.ds(w0, seq - w0)], sem)
        return (c0, c1)

    pending = [None, None]
    pending[0] = start_gather(0)
    scale = jnp.float32(1.0 / seq)
    for b in range(bpw):
        if b + 1 < bpw:
            pending[(b + 1) % 2] = start_gather(b + 1)
        for c in pending[b % 2]:
            c.wait()
        rows = bufs[b % 2]

        def body(s, accs):
            a0, a1, a2, a3 = accs
            a0 = a0 + rows[s, pl.ds(0, 16)]
            a1 = a1 + rows[s, pl.ds(16, 16)]
            a2 = a2 + rows[s, pl.ds(32, 16)]
            a3 = a3 + rows[s, pl.ds(48, 16)]
            return (a0, a1, a2, a3)

        z = jnp.zeros((16,), jnp.float32)
        a0, a1, a2, a3 = lax.fori_loop(0, seq, body, (z, z, z, z))
        xbuf[b, pl.ds(0, 16)] = a0 * scale
        xbuf[b, pl.ds(16, 16)] = a1 * scale
        xbuf[b, pl.ds(32, 16)] = a2 * scale
        xbuf[b, pl.ds(48, 16)] = a3 * scale
    pltpu.sync_copy(xbuf, x_hbm.at[pl.ds(base, bpw)])


def _pool_sc(input_ids, table):
    batch, seq = input_ids.shape
    vocab, hidden = table.shape
    bpw = batch // NW
    mesh = plsc.VectorSubcoreMesh(core_axis_name="c", subcore_axis_name="s")
    fn = pl.kernel(
        functools.partial(_pool_body, seq, hidden, bpw),
        out_type=jax.ShapeDtypeStruct((batch, hidden), jnp.float32),
        mesh=mesh,
        scratch_types=[
            pltpu.VMEM((bpw, seq), jnp.int32),
            pltpu.VMEM((seq, hidden), jnp.float32),
            pltpu.VMEM((seq, hidden), jnp.float32),
            pltpu.VMEM((bpw, hidden), jnp.float32),
            pltpu.SemaphoreType.DMA,
            pltpu.SemaphoreType.DMA,
        ],
        compiler_params=pltpu.CompilerParams(use_tc_tiling_on_sc=False),
    )
    return fn(input_ids, table)


def _mm_body(x_ref, w_ref, b_ref, out_ref):
    out_ref[...] = lax.dot_general(
        x_ref[...], w_ref[...],
        dimension_numbers=(((1,), (1,)), ((), ())),
        preferred_element_type=jnp.float32,
    ) + b_ref[...]


def _linear_tc(x, W, b, vb=2048):
    batch, hidden = x.shape
    vocab = W.shape[0]
    grid = (pl.cdiv(vocab, vb),)
    return pl.pallas_call(
        _mm_body,
        grid=grid,
        in_specs=[
            pl.BlockSpec((batch, hidden), lambda j: (0, 0)),
            pl.BlockSpec((vb, hidden), lambda j: (j, 0)),
            pl.BlockSpec((1, vb), lambda j: (0, j)),
        ],
        out_specs=pl.BlockSpec((batch, vb), lambda j: (0, j)),
        out_shape=jax.ShapeDtypeStruct((batch, vocab), jnp.float32),
        compiler_params=pltpu.CompilerParams(
            dimension_semantics=("parallel",)),
    )(x, W, b.reshape(1, vocab))


def kernel(input_ids, table, W, b):
    x = _pool_sc(input_ids, table)
    logits = _linear_tc(x, W, b)
    return (logits, x)


# speedup vs baseline: 1.4398x; 1.0050x over previous
import functools

import jax
import jax.numpy as jnp
from jax import lax
from jax.experimental import pallas as pl
from jax.experimental.pallas import tpu as pltpu
from jax.experimental.pallas import tpu_sc as plsc

NC = 2
NS = 16
NW = NC * NS


def _pool_body(seq, hidden, bpw, ids_hbm, table_hbm, x_hbm,
               idx_v, rows0, rows1, xbuf, sem0, sem1):
    half = seq // 2
    wid = lax.axis_index("s") * NC + lax.axis_index("c")
    base = wid * bpw
    pltpu.sync_copy(ids_hbm.at[pl.ds(base, bpw)], idx_v)

    bufs = (rows0, rows1)
    sems = (sem0, sem1)

    def start_gather(b):
        buf = bufs[b % 2]
        sem = sems[b % 2]
        w0 = min(128, seq)
        c0 = pltpu.async_copy(table_hbm.at[idx_v.at[b, pl.ds(0, w0)]],
                              buf.at[pl.ds(0, w0)], sem)
        c1 = pltpu.async_copy(table_hbm.at[idx_v.at[b, pl.ds(w0, seq - w0)]],
                              buf.at[pl.ds(w0, seq - w0)], sem)
        return (c0, c1)

    pending = [None, None]
    pending[0] = start_gather(0)
    scale = jnp.float32(1.0 / seq)
    for b in range(bpw):
        if b + 1 < bpw:
            pending[(b + 1) % 2] = start_gather(b + 1)
        for c in pending[b % 2]:
            c.wait()
        rows = bufs[b % 2]

        def body(s, accs):
            a0, a1, a2, a3 = accs
            a0 = a0 + rows[s, pl.ds(0, 16)]
            a1 = a1 + rows[s, pl.ds(16, 16)]
            a2 = a2 + rows[s, pl.ds(32, 16)]
            a3 = a3 + rows[s, pl.ds(48, 16)]
            return (a0, a1, a2, a3)

        z = jnp.zeros((16,), jnp.float32)
        a0, a1, a2, a3 = lax.fori_loop(0, seq, body, (z, z, z, z))
        xbuf[b, pl.ds(0, 16)] = a0 * scale
        xbuf[b, pl.ds(16, 16)] = a1 * scale
        xbuf[b, pl.ds(32, 16)] = a2 * scale
        xbuf[b, pl.ds(48, 16)] = a3 * scale
    pltpu.sync_copy(xbuf, x_hbm.at[pl.ds(base, bpw)])


def _pool_sc(input_ids, table):
    batch, seq = input_ids.shape
    vocab, hidden = table.shape
    bpw = batch // NW
    mesh = plsc.VectorSubcoreMesh(core_axis_name="c", subcore_axis_name="s")
    fn = pl.kernel(
        functools.partial(_pool_body, seq, hidden, bpw),
        out_type=jax.ShapeDtypeStruct((batch, hidden), jnp.float32),
        mesh=mesh,
        scratch_types=[
            pltpu.VMEM((bpw, seq), jnp.int32),
            pltpu.VMEM((seq, hidden), jnp.float32),
            pltpu.VMEM((seq, hidden), jnp.float32),
            pltpu.VMEM((bpw, hidden), jnp.float32),
            pltpu.SemaphoreType.DMA,
            pltpu.SemaphoreType.DMA,
        ],
        compiler_params=pltpu.CompilerParams(use_tc_tiling_on_sc=False),
    )
    return fn(input_ids, table)


def _mm_body(x_ref, w_ref, b_ref, out_ref):
    out_ref[...] = lax.dot_general(
        x_ref[...], w_ref[...],
        dimension_numbers=(((1,), (1,)), ((), ())),
        preferred_element_type=jnp.float32,
    ) + b_ref[...]


def _linear_tc(x, W, b, vb=4096):
    batch, hidden = x.shape
    vocab = W.shape[0]
    grid = (pl.cdiv(vocab, vb),)
    return pl.pallas_call(
        _mm_body,
        grid=grid,
        in_specs=[
            pl.BlockSpec((batch, hidden), lambda j: (0, 0)),
            pl.BlockSpec((vb, hidden), lambda j: (j, 0)),
            pl.BlockSpec((1, vb), lambda j: (0, j)),
        ],
        out_specs=pl.BlockSpec((batch, vb), lambda j: (0, j)),
        out_shape=jax.ShapeDtypeStruct((batch, vocab), jnp.float32),
        compiler_params=pltpu.CompilerParams(
            dimension_semantics=("parallel",)),
    )(x, W, b.reshape(1, vocab))


def kernel(input_ids, table, W, b):
    x = _pool_sc(input_ids, table)
    logits = _linear_tc(x, W, b)
    return (logits, x)


# speedup vs baseline: 1.4401x; 1.0002x over previous
import functools

import jax
import jax.numpy as jnp
from jax import lax
from jax.experimental import pallas as pl
from jax.experimental.pallas import tpu as pltpu
from jax.experimental.pallas import tpu_sc as plsc

NC = 2
NS = 16
NW = NC * NS


def _pool_body(seq, hidden, bpw, ids_hbm, table_hbm, x_hbm,
               idx_v, rows0, rows1, xbuf, sem0, sem1):
    half = seq // 2
    wid = lax.axis_index("s") * NC + lax.axis_index("c")
    base = wid * bpw
    pltpu.sync_copy(ids_hbm.at[pl.ds(base, bpw)], idx_v)

    bufs = (rows0, rows1)
    sems = (sem0, sem1)

    def start_gather(b):
        buf = bufs[b % 2]
        sem = sems[b % 2]
        w0 = min(128, seq)
        c0 = pltpu.async_copy(table_hbm.at[idx_v.at[b, pl.ds(0, w0)]],
                              buf.at[pl.ds(0, w0)], sem)
        c1 = pltpu.async_copy(table_hbm.at[idx_v.at[b, pl.ds(w0, seq - w0)]],
                              buf.at[pl.ds(w0, seq - w0)], sem)
        return (c0, c1)

    pending = [None, None]
    pending[0] = start_gather(0)
    scale = jnp.float32(1.0 / seq)
    for b in range(bpw):
        if b + 1 < bpw:
            pending[(b + 1) % 2] = start_gather(b + 1)
        for c in pending[b % 2]:
            c.wait()
        rows = bufs[b % 2]

        unroll = 8
        assert seq % unroll == 0

        def body(i, accs):
            a0, a1, a2, a3 = accs
            s0 = i * unroll
            for u in range(unroll):
                a0 = a0 + rows[s0 + u, pl.ds(0, 16)]
                a1 = a1 + rows[s0 + u, pl.ds(16, 16)]
                a2 = a2 + rows[s0 + u, pl.ds(32, 16)]
                a3 = a3 + rows[s0 + u, pl.ds(48, 16)]
            return (a0, a1, a2, a3)

        z = jnp.zeros((16,), jnp.float32)
        a0, a1, a2, a3 = lax.fori_loop(0, seq // unroll, body, (z, z, z, z))
        xbuf[b, pl.ds(0, 16)] = a0 * scale
        xbuf[b, pl.ds(16, 16)] = a1 * scale
        xbuf[b, pl.ds(32, 16)] = a2 * scale
        xbuf[b, pl.ds(48, 16)] = a3 * scale
    pltpu.sync_copy(xbuf, x_hbm.at[pl.ds(base, bpw)])


def _pool_sc(input_ids, table):
    batch, seq = input_ids.shape
    vocab, hidden = table.shape
    bpw = batch // NW
    mesh = plsc.VectorSubcoreMesh(core_axis_name="c", subcore_axis_name="s")
    fn = pl.kernel(
        functools.partial(_pool_body, seq, hidden, bpw),
        out_type=jax.ShapeDtypeStruct((batch, hidden), jnp.float32),
        mesh=mesh,
        scratch_types=[
            pltpu.VMEM((bpw, seq), jnp.int32),
            pltpu.VMEM((seq, hidden), jnp.float32),
            pltpu.VMEM((seq, hidden), jnp.float32),
            pltpu.VMEM((bpw, hidden), jnp.float32),
            pltpu.SemaphoreType.DMA,
            pltpu.SemaphoreType.DMA,
        ],
        compiler_params=pltpu.CompilerParams(use_tc_tiling_on_sc=False),
    )
    return fn(input_ids, table)


def _mm_body(x_ref, w_ref, b_ref, out_ref):
    out_ref[...] = lax.dot_general(
        x_ref[...], w_ref[...],
        dimension_numbers=(((1,), (1,)), ((), ())),
        preferred_element_type=jnp.float32,
    ) + b_ref[...]


def _linear_tc(x, W, b, vb=4096):
    batch, hidden = x.shape
    vocab = W.shape[0]
    grid = (pl.cdiv(vocab, vb),)
    return pl.pallas_call(
        _mm_body,
        grid=grid,
        in_specs=[
            pl.BlockSpec((batch, hidden), lambda j: (0, 0)),
            pl.BlockSpec((vb, hidden), lambda j: (j, 0)),
            pl.BlockSpec((1, vb), lambda j: (0, j)),
        ],
        out_specs=pl.BlockSpec((batch, vb), lambda j: (0, j)),
        out_shape=jax.ShapeDtypeStruct((batch, vocab), jnp.float32),
        compiler_params=pltpu.CompilerParams(
            dimension_semantics=("parallel",)),
    )(x, W, b.reshape(1, vocab))


def kernel(input_ids, table, W, b):
    x = _pool_sc(input_ids, table)
    logits = _linear_tc(x, W, b)
    return (logits, x)


# speedup vs baseline: 1.4447x; 1.0032x over previous
import functools

import jax
import jax.numpy as jnp
from jax import lax
from jax.experimental import pallas as pl
from jax.experimental.pallas import tpu as pltpu
from jax.experimental.pallas import tpu_sc as plsc

NC = 2
NS = 16
NW = NC * NS


NBUF = 4


def _pool_body(seq, hidden, bpw, ids_hbm, table_hbm, x_hbm,
               idx_v, rows_bufs, xbuf, sems):
    wid = lax.axis_index("s") * NC + lax.axis_index("c")
    base = wid * bpw
    pltpu.sync_copy(ids_hbm.at[pl.ds(base, bpw)], idx_v)

    def start_gather(b):
        buf = rows_bufs[b % NBUF]
        sem = sems[b % NBUF]
        w0 = min(128, seq)
        c0 = pltpu.async_copy(table_hbm.at[idx_v.at[b, pl.ds(0, w0)]],
                              buf.at[pl.ds(0, w0)], sem)
        c1 = pltpu.async_copy(table_hbm.at[idx_v.at[b, pl.ds(w0, seq - w0)]],
                              buf.at[pl.ds(w0, seq - w0)], sem)
        return (c0, c1)

    pending = [None] * NBUF
    for b in range(NBUF - 1):
        pending[b] = start_gather(b)
    scale = jnp.float32(1.0 / seq)
    for b in range(bpw):
        if b + NBUF - 1 < bpw:
            pending[(b + NBUF - 1) % NBUF] = start_gather(b + NBUF - 1)
        for c in pending[b % NBUF]:
            c.wait()
        rows = rows_bufs[b % NBUF]

        unroll = 8
        assert seq % unroll == 0

        def body(i, accs):
            a0, a1, a2, a3 = accs
            s0 = i * unroll
            for u in range(unroll):
                a0 = a0 + rows[s0 + u, pl.ds(0, 16)]
                a1 = a1 + rows[s0 + u, pl.ds(16, 16)]
                a2 = a2 + rows[s0 + u, pl.ds(32, 16)]
                a3 = a3 + rows[s0 + u, pl.ds(48, 16)]
            return (a0, a1, a2, a3)

        z = jnp.zeros((16,), jnp.float32)
        a0, a1, a2, a3 = lax.fori_loop(0, seq // unroll, body, (z, z, z, z))
        xbuf[b, pl.ds(0, 16)] = a0 * scale
        xbuf[b, pl.ds(16, 16)] = a1 * scale
        xbuf[b, pl.ds(32, 16)] = a2 * scale
        xbuf[b, pl.ds(48, 16)] = a3 * scale
    pltpu.sync_copy(xbuf, x_hbm.at[pl.ds(base, bpw)])


def _pool_sc(input_ids, table):
    batch, seq = input_ids.shape
    vocab, hidden = table.shape
    bpw = batch // NW
    mesh = plsc.VectorSubcoreMesh(core_axis_name="c", subcore_axis_name="s")

    def body(ids_hbm, table_hbm, x_hbm, idx_v, *rest):
        rows_bufs = rest[:NBUF]
        xbuf = rest[NBUF]
        sems = rest[NBUF + 1:]
        _pool_body(seq, hidden, bpw, ids_hbm, table_hbm, x_hbm,
                   idx_v, rows_bufs, xbuf, sems)

    fn = pl.kernel(
        body,
        out_type=jax.ShapeDtypeStruct((batch, hidden), jnp.float32),
        mesh=mesh,
        scratch_types=(
            [pltpu.VMEM((bpw, seq), jnp.int32)]
            + [pltpu.VMEM((seq, hidden), jnp.float32) for _ in range(NBUF)]
            + [pltpu.VMEM((bpw, hidden), jnp.float32)]
            + [pltpu.SemaphoreType.DMA for _ in range(NBUF)]
        ),
        compiler_params=pltpu.CompilerParams(use_tc_tiling_on_sc=False),
    )
    return fn(input_ids, table)


def _mm_body(x_ref, w_ref, b_ref, out_ref):
    out_ref[...] = lax.dot_general(
        x_ref[...], w_ref[...],
        dimension_numbers=(((1,), (1,)), ((), ())),
        preferred_element_type=jnp.float32,
    ) + b_ref[...]


def _linear_tc(x, W, b, vb=4096):
    batch, hidden = x.shape
    vocab = W.shape[0]
    grid = (pl.cdiv(vocab, vb),)
    return pl.pallas_call(
        _mm_body,
        grid=grid,
        in_specs=[
            pl.BlockSpec((batch, hidden), lambda j: (0, 0)),
            pl.BlockSpec((vb, hidden), lambda j: (j, 0)),
            pl.BlockSpec((1, vb), lambda j: (0, j)),
        ],
        out_specs=pl.BlockSpec((batch, vb), lambda j: (0, j)),
        out_shape=jax.ShapeDtypeStruct((batch, vocab), jnp.float32),
        compiler_params=pltpu.CompilerParams(
            dimension_semantics=("parallel",)),
    )(x, W, b.reshape(1, vocab))


def kernel(input_ids, table, W, b):
    x = _pool_sc(input_ids, table)
    logits = _linear_tc(x, W, b)
    return (logits, x)
